# baseline (speedup 1.0000x reference)
import functools

import jax
import jax.numpy as jnp
from jax import lax
from jax.experimental import pallas as pl
from jax.experimental.pallas import tpu as pltpu
from jax.experimental.pallas import tpu_sc as plsc

_NC = 2
_NS = 16
_NW = _NC * _NS
_CHUNK = 128


@functools.lru_cache(maxsize=None)
def _build(n_rows: int, embed: int, n_chunks: int):
    mesh = plsc.VectorSubcoreMesh(core_axis_name="c", subcore_axis_name="s")

    @functools.partial(
        pl.kernel,
        out_type=jax.ShapeDtypeStruct((n_rows, embed), jnp.float32),
        mesh=mesh,
        compiler_params=pltpu.CompilerParams(use_tc_tiling_on_sc=False),
        scratch_types=[
            pltpu.VMEM((n_chunks, _CHUNK), jnp.int32),
            pltpu.VMEM((_CHUNK, embed), jnp.float32),
            pltpu.SemaphoreType.DMA,
        ],
    )
    def gather(table_hbm, idx_hbm, out_hbm, idx_v, rows_v, sem):
        wid = lax.axis_index("s") * _NC + lax.axis_index("c")
        pltpu.sync_copy(idx_hbm.at[wid], idx_v)
        base = wid * (n_chunks * _CHUNK)

        def chunk(j, carry):
            pltpu.async_copy(table_hbm.at[idx_v.at[j]], rows_v, sem).wait()
            pltpu.sync_copy(rows_v, out_hbm.at[pl.ds(base + j * _CHUNK, _CHUNK)])
            return carry

        lax.fori_loop(0, n_chunks, chunk, 0)

    return gather


def kernel(type_index, entity_table):
    b, h = type_index.shape
    embed = entity_table.shape[1]
    n_rows = b * h
    n_chunks = n_rows // (_NW * _CHUNK)
    idx = type_index.reshape(_NW, n_chunks, _CHUNK)
    out = _build(n_rows, embed, n_chunks)(entity_table, idx)
    return out.reshape(b, h, embed)


# speedup vs baseline: 1.0359x; 1.0359x over previous
import functools

import jax
import jax.numpy as jnp
from jax import lax
from jax.experimental import pallas as pl
from jax.experimental.pallas import tpu as pltpu
from jax.experimental.pallas import tpu_sc as plsc

_NC = 2
_NS = 16
_NW = _NC * _NS
_CHUNK = 128


@functools.lru_cache(maxsize=None)
def _build(n_rows: int, embed: int, n_chunks: int):
    mesh = plsc.VectorSubcoreMesh(core_axis_name="c", subcore_axis_name="s")

    @functools.partial(
        pl.kernel,
        out_type=jax.ShapeDtypeStruct((n_rows, embed), jnp.float32),
        mesh=mesh,
        compiler_params=pltpu.CompilerParams(use_tc_tiling_on_sc=False),
        scratch_types=[
            pltpu.VMEM((n_chunks, _CHUNK), jnp.int32),
            pltpu.VMEM((_CHUNK, embed), jnp.float32),
            pltpu.VMEM((_CHUNK, embed), jnp.float32),
            pltpu.SemaphoreType.DMA,
            pltpu.SemaphoreType.DMA,
        ],
    )
    def gather(table_hbm, idx_hbm, out_hbm, idx_v, rows_a, rows_b, sem_a, sem_b):
        wid = lax.axis_index("s") * _NC + lax.axis_index("c")
        pltpu.sync_copy(idx_hbm.at[wid], idx_v)
        base = wid * (n_chunks * _CHUNK)
        bufs = (rows_a, rows_b)
        sems = (sem_a, sem_b)

        def start(j, slot):
            pltpu.make_async_copy(
                table_hbm.at[idx_v.at[j]], bufs[slot], sems[slot]
            ).start()

        def finish(j, slot):
            pltpu.make_async_copy(
                table_hbm.at[idx_v.at[j]], bufs[slot], sems[slot]
            ).wait()
            pltpu.sync_copy(
                bufs[slot], out_hbm.at[pl.ds(base + j * _CHUNK, _CHUNK)]
            )

        start(0, 0)

        @pl.loop(0, n_chunks, step=2)
        def pair(j):
            start(j + 1, 1)
            finish(j, 0)

            @pl.when(j + 2 < n_chunks)
            def _():
                start(j + 2, 0)

            finish(j + 1, 1)

    return gather


def kernel(type_index, entity_table):
    b, h = type_index.shape
    embed = entity_table.shape[1]
    n_rows = b * h
    n_chunks = n_rows // (_NW * _CHUNK)
    idx = type_index.reshape(_NW, n_chunks, _CHUNK)
    out = _build(n_rows, embed, n_chunks)(entity_table, idx)
    return out.reshape(b, h, embed)


# speedup vs baseline: 1.0455x; 1.0092x over previous
import functools

import jax
import jax.numpy as jnp
from jax import lax
from jax.experimental import pallas as pl
from jax.experimental.pallas import tpu as pltpu
from jax.experimental.pallas import tpu_sc as plsc

_NC = 2
_NS = 16
_NW = _NC * _NS
_CHUNK = 128
_NBUF = 10


@functools.lru_cache(maxsize=None)
def _build(n_rows: int, embed: int, n_chunks: int):
    mesh = plsc.VectorSubcoreMesh(core_axis_name="c", subcore_axis_name="s")

    @functools.partial(
        pl.kernel,
        out_type=jax.ShapeDtypeStruct((n_rows, embed), jnp.float32),
        mesh=mesh,
        compiler_params=pltpu.CompilerParams(use_tc_tiling_on_sc=False),
        scratch_types=[
            pltpu.VMEM((n_chunks, _CHUNK), jnp.int32),
            pltpu.VMEM((_NBUF, _CHUNK, embed), jnp.float32),
        ]
        + [pltpu.SemaphoreType.DMA] * _NBUF,
    )
    def gather(table_hbm, idx_hbm, out_hbm, idx_v, rows_v, *sems):
        wid = lax.axis_index("s") * _NC + lax.axis_index("c")
        pltpu.sync_copy(idx_hbm.at[wid], idx_v)
        base = wid * (n_chunks * _CHUNK)

        def start(j, slot):
            pltpu.make_async_copy(
                table_hbm.at[idx_v.at[j]], rows_v.at[slot], sems[slot]
            ).start()

        def finish(j, slot):
            pltpu.make_async_copy(
                table_hbm.at[idx_v.at[j]], rows_v.at[slot], sems[slot]
            ).wait()
            pltpu.sync_copy(
                rows_v.at[slot], out_hbm.at[pl.ds(base + j * _CHUNK, _CHUNK)]
            )

        for j in range(_NBUF):
            start(j, j)

        @pl.loop(0, n_chunks, step=_NBUF)
        def ring(j):
            for b in range(_NBUF):
                finish(j + b, b)

                @pl.when(j + b + _NBUF < n_chunks)
                def _():
                    start(j + b + _NBUF, b)

    return gather


def kernel(type_index, entity_table):
    b, h = type_index.shape
    embed = entity_table.shape[1]
    n_rows = b * h
    n_chunks = n_rows // (_NW * _CHUNK)
    idx = type_index.reshape(_NW, n_chunks, _CHUNK)
    out = _build(n_rows, embed, n_chunks)(entity_table, idx)
    return out.reshape(b, h, embed)
